# baseline (speedup 1.0000x reference)
import functools

import jax
import jax.numpy as jnp
from jax import lax
from jax.experimental import pallas as pl
from jax.experimental.pallas import tpu as pltpu
from jax.experimental.pallas import tpu_sc as plsc

N = 10000
NP = 10240
E = 160000
EP = 163840
EC = 1280
G = 128
GP = 256
D_IN = 256
H = 512
CB = 128
NBLK = 4
NC = 2
NS = 16
RB = 640

_f32 = jnp.float32
_i32 = jnp.int32

_sc_mesh = plsc.VectorSubcoreMesh(core_axis_name="c", subcore_axis_name="s")


def _fill(ref, n16, value):
    v = jnp.full((16,), value, dtype=ref.dtype)

    def body(i, _):
        ref[pl.ds(i * 16, 16)] = v
        return 0

    lax.fori_loop(0, n16, body, 0)


def _fill2d(ref, rows, value):
    cols = ref.shape[1]
    v = jnp.full((16,), value, dtype=ref.dtype)

    def body(i, _):
        for t in range(cols // 16):
            ref[i, pl.ds(t * 16, 16)] = v
        return 0

    lax.fori_loop(0, rows, body, 0)


def _deg_body(d2, w2, deg_out, dacc, ibuf, wbuf, zb):
    c = lax.axis_index("c")
    s = lax.axis_index("s")
    _fill(zb, 40, 0.0)
    pltpu.sync_copy(zb, dacc.at[pl.ds(s * 640, 640)])
    plsc.subcore_barrier()
    first = c * 640 + s * 40
    pltpu.sync_copy(d2.at[pl.ds(first, 40)], ibuf)
    pltpu.sync_copy(w2.at[pl.ds(first, 40)], wbuf)

    def chunk(j, _):
        pltpu.sync_copy(wbuf.at[j], dacc.at[ibuf.at[j]], add=True)
        return 0

    lax.fori_loop(0, 40, chunk, 0)
    plsc.subcore_barrier()
    pltpu.sync_copy(dacc.at[pl.ds(s * 640, 640)],
                    deg_out.at[c, pl.ds(s * 640, 640)])


_deg_call = functools.partial(
    pl.kernel,
    out_type=jax.ShapeDtypeStruct((NC, NP), _f32),
    mesh=_sc_mesh,
    scratch_types=[
        pltpu.VMEM_SHARED((NP,), _f32),
        pltpu.VMEM((40, 128), _i32),
        pltpu.VMEM((40, 128), _f32),
        pltpu.VMEM((640,), _f32),
    ],
)(_deg_body)


def _agg_body(hp, edata, w16, agg_out, acc, rows0, rows1, gbuf0, gbuf1,
              ering, wbc0, wbc1, gsem0, gsem1, wsem0, wsem1, ssem0, ssem1,
              isem0, isem1, isem2, isem3):
    c = lax.axis_index("c")
    s = lax.axis_index("s")
    rows = (rows0, rows1)
    gbufs = (gbuf0, gbuf1)
    wbcs = (wbc0, wbc1)
    gsems = (gsem0, gsem1)
    wsems = (wsem0, wsem1)
    ssems = (ssem0, ssem1)
    isems = (isem0, isem1, isem2, isem3)

    def fire_idx(first, j, slot, isem):
        pltpu.async_copy(edata.at[first + j], ering.at[slot], isem)

    def drain_idx(isem):
        pltpu.make_async_copy(edata.at[0], ering.at[0], isem).wait()

    def fire_rows(first, j, base, slot, p):
        def mk(tt, _):
            gbufs[p][pl.ds(tt * 16, 16)] = (
                ering[slot, pl.ds(tt * 16, 16)] + base)
            return 0

        lax.fori_loop(0, 8, mk, 0)
        pltpu.async_copy(hp.at[gbufs[p]], rows[p], gsems[p])
        pltpu.async_copy(w16.at[first + j], wbcs[p], wsems[p])

    def drain_rows(p):
        pltpu.make_async_copy(hp.at[pl.ds(0, 128)], rows[p],
                              gsems[p]).wait()
        pltpu.make_async_copy(w16.at[0], wbcs[p], wsems[p]).wait()

    def drain_scat(p):
        pltpu.make_async_copy(hp.at[pl.ds(0, 128)], rows[p],
                              ssems[p]).wait()

    def scale(rr, wbc, lo, hi):
        def body(i, _):
            for u in range(2):
                e = 2 * i + u
                wv = wbc[pl.ds(e * 16, 16)]
                for tt in range(CB // 16):
                    rr[e, pl.ds(tt * 16, 16)] = (
                        rr[e, pl.ds(tt * 16, 16)] * wv)
            return 0

        lax.fori_loop(lo // 2, hi // 2, body, 0)

    for blk_i in range(2):
        blk = c * 2 + blk_i
        base = blk * NP
        _fill2d(rows0, 128, 0.0)
        for k in range(5):
            pltpu.sync_copy(rows0, acc.at[pl.ds(s * 640 + k * 128, 128)])
        plsc.subcore_barrier()

        for half in range(2):
            first = s * 80 + half * 40
            fire_idx(first, 0, 0, isem0)
            fire_idx(first, 1, 1, isem1)
            drain_idx(isem0)
            fire_rows(first, 0, base, 0, 0)

            def quad(k, _):
                for t in range(4):
                    j = 4 * k + t
                    p = t & 1
                    slot_j = lax.rem(j, 4)

                    drain_rows(p)
                    scale(rows[p], wbcs[p], 0, 64)
                    if t == 0:
                        @pl.when(k > 0)
                        def _():
                            drain_scat(1 - p)
                    else:
                        drain_scat(1 - p)
                    if t == 3:
                        @pl.when(k < 9)
                        def _():
                            drain_idx(isems[(t + 1) & 3])
                            fire_rows(first, j + 1, base, (t + 1) & 3, 1 - p)
                            fire_idx(first, j + 2, (t + 2) & 3,
                                     isems[(t + 2) & 3])
                    else:
                        drain_idx(isems[(t + 1) & 3])
                        fire_rows(first, j + 1, base, (t + 1) & 3, 1 - p)

                        if t == 2:
                            @pl.when(k < 9)
                            def _():
                                fire_idx(first, j + 2, (t + 2) & 3,
                                         isems[(t + 2) & 3])
                        else:
                            fire_idx(first, j + 2, (t + 2) & 3,
                                     isems[(t + 2) & 3])
                    scale(rows[p], wbcs[p], 64, 128)
                    pltpu.async_copy(
                        rows[p], acc.at[ering.at[slot_j, pl.ds(128, 128)]],
                        ssems[p], add=True)
                return 0

            lax.fori_loop(0, 10, quad, 0)
            drain_scat(1)
        plsc.subcore_barrier()
        pltpu.sync_copy(acc.at[pl.ds(s * 640, 640)],
                        agg_out.at[pl.ds(base + s * 640, 640)])
        plsc.subcore_barrier()


_agg_call = functools.partial(
    pl.kernel,
    out_type=jax.ShapeDtypeStruct((NBLK * NP, CB), _f32),
    mesh=_sc_mesh,
    scratch_types=[
        pltpu.VMEM_SHARED((NP, CB), _f32),
        pltpu.VMEM((128, CB), _f32),
        pltpu.VMEM((128, CB), _f32),
        pltpu.VMEM((128,), _i32),
        pltpu.VMEM((128,), _i32),
        pltpu.VMEM((4, 256), _i32),
        pltpu.VMEM((2048,), _f32),
        pltpu.VMEM((2048,), _f32),
        pltpu.SemaphoreType.DMA,
        pltpu.SemaphoreType.DMA,
        pltpu.SemaphoreType.DMA,
        pltpu.SemaphoreType.DMA,
        pltpu.SemaphoreType.DMA,
        pltpu.SemaphoreType.DMA,
        pltpu.SemaphoreType.DMA,
        pltpu.SemaphoreType.DMA,
        pltpu.SemaphoreType.DMA,
        pltpu.SemaphoreType.DMA,
    ],
)(_agg_body)


def _pool_body(z4, b2d, sums_out, cnts_out, sa0, sa1, sa2, sa3, cacc, ibuf,
               rows, onesb, zbp, zbc):
    c = lax.axis_index("c")
    s = lax.axis_index("s")
    wid = c * NS + s
    saccs = (sa0, sa1, sa2, sa3)
    _fill(onesb, 8, 1.0)
    _fill2d(zbp, 16, 0.0)
    for q in range(4):
        pltpu.sync_copy(zbp, saccs[q].at[pl.ds(s * 16, 16)])

    @pl.when(s == 0)
    def _():
        _fill(zbc, 16, 0.0)
        pltpu.sync_copy(zbc, cacc)

    plsc.subcore_barrier()

    def chunk(k, _):
        cid = wid + 32 * k

        @pl.when(cid < 80)
        def _():
            pltpu.sync_copy(b2d.at[cid], ibuf.at[k])
            for q in range(4):
                pltpu.sync_copy(
                    z4.at[pl.ds(cid * 128, 128), pl.ds(q * 128, 128)], rows)
                pltpu.sync_copy(rows, saccs[q].at[ibuf.at[k]], add=True)
            pltpu.sync_copy(onesb, cacc.at[ibuf.at[k]], add=True)

        return 0

    lax.fori_loop(0, 3, chunk, 0)
    plsc.subcore_barrier()
    for q in range(4):
        pltpu.sync_copy(saccs[q].at[pl.ds(s * 16, 16)],
                        sums_out.at[c, q, pl.ds(s * 16, 16)])

    @pl.when(s == 0)
    def _():
        pltpu.sync_copy(cacc, cnts_out.at[c])


_pool_call = functools.partial(
    pl.kernel,
    out_type=(jax.ShapeDtypeStruct((NC, NBLK, GP, CB), _f32),
              jax.ShapeDtypeStruct((NC, GP), _f32)),
    mesh=_sc_mesh,
    scratch_types=[
        pltpu.VMEM_SHARED((GP, CB), _f32),
        pltpu.VMEM_SHARED((GP, CB), _f32),
        pltpu.VMEM_SHARED((GP, CB), _f32),
        pltpu.VMEM_SHARED((GP, CB), _f32),
        pltpu.VMEM_SHARED((GP,), _f32),
        pltpu.VMEM((3, 128), _i32),
        pltpu.VMEM((128, CB), _f32),
        pltpu.VMEM((128,), _f32),
        pltpu.VMEM((16, CB), _f32),
        pltpu.VMEM((GP,), _f32),
    ],
)(_pool_body)


def _mm_kernel(x_ref, w_ref, g0_ref, g1_ref, o_ref):
    dinv = lax.rsqrt(g0_ref[...] + g1_ref[...] + 1.0)
    h = jnp.dot(x_ref[...], w_ref[...], preferred_element_type=_f32)
    o_ref[...] = h * dinv


def _mm(xp, w, g0, g1):
    din = xp.shape[1]
    return pl.pallas_call(
        _mm_kernel,
        grid=(NP // RB, NBLK),
        in_specs=[
            pl.BlockSpec((RB, din), lambda r, c: (r, 0)),
            pl.BlockSpec((din, CB), lambda r, c: (0, c)),
            pl.BlockSpec((RB, 1), lambda r, c: (r, 0)),
            pl.BlockSpec((RB, 1), lambda r, c: (r, 0)),
        ],
        out_specs=pl.BlockSpec((RB, CB), lambda r, c: (c * (NP // RB) + r, 0)),
        out_shape=jax.ShapeDtypeStruct((NBLK * NP, CB), _f32),
    )(xp, w, g0, g1)


def _comb_kernel(a_ref, h_ref, g0_ref, g1_ref, b_ref, o_ref):
    dinv = lax.rsqrt(g0_ref[...] + g1_ref[...] + 1.0)
    o_ref[...] = jnp.maximum(
        dinv * (a_ref[...] + h_ref[...]) + b_ref[0:1, :], 0.0)


def _comb(agg, hp, g0, g1, b4):
    return pl.pallas_call(
        _comb_kernel,
        grid=(NP // RB, NBLK),
        in_specs=[
            pl.BlockSpec((RB, CB), lambda r, c: (c * (NP // RB) + r, 0)),
            pl.BlockSpec((RB, CB), lambda r, c: (c * (NP // RB) + r, 0)),
            pl.BlockSpec((RB, 1), lambda r, c: (r, 0)),
            pl.BlockSpec((RB, 1), lambda r, c: (r, 0)),
            pl.BlockSpec((8, CB), lambda r, c: (c, 0)),
        ],
        out_specs=pl.BlockSpec((RB, CB), lambda r, c: (r, c)),
        out_shape=jax.ShapeDtypeStruct((NP, H), _f32),
    )(agg, hp, g0, g1, b4)


def _fin_kernel(s0_ref, s1_ref, c0_ref, c1_ref, wc_ref, bc_ref, o_ref):
    cnt = jnp.maximum(c0_ref[...] + c1_ref[...], 1.0)
    pooled = (s0_ref[...] + s1_ref[...]) / cnt
    o_ref[...] = jnp.dot(pooled, wc_ref[...],
                         preferred_element_type=_f32) + bc_ref[...]


def _fin(s0, s1, c0, c1, wc, bc1):
    return pl.pallas_call(
        _fin_kernel,
        out_shape=jax.ShapeDtypeStruct((G, 4), _f32),
    )(s0, s1, c0, c1, wc, bc1)


def kernel(x, edge_index, edge_attr, batch, W1, b1, W2, b2, W3, b3, Wc, bc):
    src = edge_index[0]
    dst = edge_index[1]
    pad = EP - E
    s2 = jnp.concatenate([src, jnp.zeros((pad,), _i32)]).reshape(EC, 128)

    d2 = jnp.concatenate([dst, jnp.full((pad,), NP - 1, _i32)]).reshape(EC, 128)
    wp = jnp.concatenate([edge_attr, jnp.zeros((pad,), _f32)])
    w2 = wp.reshape(EC, 128)
    edata = jnp.concatenate([s2, d2], axis=1)
    w16 = jnp.broadcast_to(wp[:, None], (EP, 16)).reshape(EC, 2048)
    xp = jnp.concatenate([x, jnp.zeros((NP - N, D_IN), _f32)])
    b2d = jnp.concatenate([batch, jnp.full((NP - N,), G, _i32)]).reshape(80, 128)

    deg = _deg_call(d2, w2)
    g0 = deg[0].reshape(NP, 1)
    g1 = deg[1].reshape(NP, 1)

    def _brd(b):
        return jnp.broadcast_to(b.reshape(NBLK, 1, CB),
                                (NBLK, 8, CB)).reshape(NBLK * 8, CB)

    b1r = _brd(b1)
    b2r = _brd(b2)
    b3r = _brd(b3)

    hp1 = _mm(xp, W1, g0, g1)
    ag1 = _agg_call(hp1, edata, w16)
    z2 = _comb(ag1, hp1, g0, g1, b1r)

    hp2 = _mm(z2, W2, g0, g1)
    ag2 = _agg_call(hp2, edata, w16)
    z3 = _comb(ag2, hp2, g0, g1, b2r)

    hp3 = _mm(z3, W3, g0, g1)
    ag3 = _agg_call(hp3, edata, w16)
    z4 = _comb(ag3, hp3, g0, g1, b3r)

    sums, cnts = _pool_call(z4, b2d)
    sums = sums.transpose(0, 2, 1, 3).reshape(NC, GP, H)
    s0 = sums[0, :G]
    s1 = sums[1, :G]
    c0 = cnts[0, :G].reshape(G, 1)
    c1 = cnts[1, :G].reshape(G, 1)
    return _fin(s0, s1, c0, c1, Wc, bc.reshape(1, 4))


# speedup vs baseline: 1.0102x; 1.0102x over previous
import functools

import jax
import jax.numpy as jnp
from jax import lax
from jax.experimental import pallas as pl
from jax.experimental.pallas import tpu as pltpu
from jax.experimental.pallas import tpu_sc as plsc

N = 10000
NP = 10240
E = 160000
EP = 163840
EC = 1280
G = 128
GP = 256
D_IN = 256
H = 512
CB = 128
NBLK = 4
NC = 2
NS = 16
RB = 640

_f32 = jnp.float32
_i32 = jnp.int32

_sc_mesh = plsc.VectorSubcoreMesh(core_axis_name="c", subcore_axis_name="s")


def _fill(ref, n16, value):
    v = jnp.full((16,), value, dtype=ref.dtype)

    def body(i, _):
        ref[pl.ds(i * 16, 16)] = v
        return 0

    lax.fori_loop(0, n16, body, 0)


def _fill2d(ref, rows, value):
    cols = ref.shape[1]
    v = jnp.full((16,), value, dtype=ref.dtype)

    def body(i, _):
        for t in range(cols // 16):
            ref[i, pl.ds(t * 16, 16)] = v
        return 0

    lax.fori_loop(0, rows, body, 0)


def _deg_body(d2, w2, deg_out, dacc, ibuf, wbuf, zb):
    c = lax.axis_index("c")
    s = lax.axis_index("s")
    _fill(zb, 40, 0.0)
    pltpu.sync_copy(zb, dacc.at[pl.ds(s * 640, 640)])
    plsc.subcore_barrier()
    first = c * 640 + s * 40
    pltpu.sync_copy(d2.at[pl.ds(first, 40)], ibuf)
    pltpu.sync_copy(w2.at[pl.ds(first, 40)], wbuf)

    def chunk(j, _):
        pltpu.sync_copy(wbuf.at[j], dacc.at[ibuf.at[j]], add=True)
        return 0

    lax.fori_loop(0, 40, chunk, 0)
    plsc.subcore_barrier()
    pltpu.sync_copy(dacc.at[pl.ds(s * 640, 640)],
                    deg_out.at[c, pl.ds(s * 640, 640)])


_deg_call = functools.partial(
    pl.kernel,
    out_type=jax.ShapeDtypeStruct((NC, NP), _f32),
    mesh=_sc_mesh,
    scratch_types=[
        pltpu.VMEM_SHARED((NP,), _f32),
        pltpu.VMEM((40, 128), _i32),
        pltpu.VMEM((40, 128), _f32),
        pltpu.VMEM((640,), _f32),
    ],
)(_deg_body)


def _agg_body(hp, edata, w16, agg_out, acc, rows0, rows1, ering, wbc0,
              wbc1, gsem0, gsem1, wsem0, wsem1, ssem0, ssem1, isem0, isem1,
              isem2, isem3):
    c = lax.axis_index("c")
    s = lax.axis_index("s")
    rows = (rows0, rows1)
    wbcs = (wbc0, wbc1)
    gsems = (gsem0, gsem1)
    wsems = (wsem0, wsem1)
    ssems = (ssem0, ssem1)
    isems = (isem0, isem1, isem2, isem3)

    def fire_idx(blk, first, j, slot, isem):
        pltpu.async_copy(edata.at[blk, first + j], ering.at[slot], isem)

    def drain_idx(isem):
        pltpu.make_async_copy(edata.at[0, 0], ering.at[0], isem).wait()

    def fire_rows(first, j, slot, p):
        pltpu.async_copy(hp.at[ering.at[slot, pl.ds(0, 128)]], rows[p],
                         gsems[p])
        pltpu.async_copy(w16.at[first + j], wbcs[p], wsems[p])

    def drain_rows(p):
        pltpu.make_async_copy(hp.at[pl.ds(0, 128)], rows[p],
                              gsems[p]).wait()
        pltpu.make_async_copy(w16.at[0], wbcs[p], wsems[p]).wait()

    def drain_scat(p):
        pltpu.make_async_copy(hp.at[pl.ds(0, 128)], rows[p],
                              ssems[p]).wait()

    def scale(rr, wbc, lo, hi):
        def body(i, _):
            for u in range(2):
                e = 2 * i + u
                wv = wbc[pl.ds(e * 16, 16)]
                for tt in range(CB // 16):
                    rr[e, pl.ds(tt * 16, 16)] = (
                        rr[e, pl.ds(tt * 16, 16)] * wv)
            return 0

        lax.fori_loop(lo // 2, hi // 2, body, 0)

    for blk_i in range(2):
        blk = c * 2 + blk_i
        base = blk * NP
        _fill2d(rows0, 128, 0.0)
        for k in range(5):
            pltpu.sync_copy(rows0, acc.at[pl.ds(s * 640 + k * 128, 128)])
        plsc.subcore_barrier()

        for half in range(2):
            first = s * 80 + half * 40
            fire_idx(blk, first, 0, 0, isem0)
            fire_idx(blk, first, 1, 1, isem1)
            drain_idx(isem0)
            fire_rows(first, 0, 0, 0)

            def quad(k, _):
                for t in range(4):
                    j = 4 * k + t
                    p = t & 1
                    slot_j = lax.rem(j, 4)

                    drain_rows(p)
                    scale(rows[p], wbcs[p], 0, 64)
                    if t == 0:
                        @pl.when(k > 0)
                        def _():
                            drain_scat(1 - p)
                    else:
                        drain_scat(1 - p)
                    if t == 3:
                        @pl.when(k < 9)
                        def _():
                            drain_idx(isems[(t + 1) & 3])
                            fire_rows(first, j + 1, (t + 1) & 3, 1 - p)
                            fire_idx(blk, first, j + 2, (t + 2) & 3,
                                     isems[(t + 2) & 3])
                    else:
                        drain_idx(isems[(t + 1) & 3])
                        fire_rows(first, j + 1, (t + 1) & 3, 1 - p)

                        if t == 2:
                            @pl.when(k < 9)
                            def _():
                                fire_idx(blk, first, j + 2, (t + 2) & 3,
                                         isems[(t + 2) & 3])
                        else:
                            fire_idx(blk, first, j + 2, (t + 2) & 3,
                                     isems[(t + 2) & 3])
                    scale(rows[p], wbcs[p], 64, 128)
                    pltpu.async_copy(
                        rows[p], acc.at[ering.at[slot_j, pl.ds(128, 128)]],
                        ssems[p], add=True)
                return 0

            lax.fori_loop(0, 10, quad, 0)
            drain_scat(1)
        plsc.subcore_barrier()
        pltpu.sync_copy(acc.at[pl.ds(s * 640, 640)],
                        agg_out.at[pl.ds(base + s * 640, 640)])
        plsc.subcore_barrier()


_agg_call = functools.partial(
    pl.kernel,
    out_type=jax.ShapeDtypeStruct((NBLK * NP, CB), _f32),
    mesh=_sc_mesh,
    scratch_types=[
        pltpu.VMEM_SHARED((NP, CB), _f32),
        pltpu.VMEM((128, CB), _f32),
        pltpu.VMEM((128, CB), _f32),
        pltpu.VMEM((4, 256), _i32),
        pltpu.VMEM((2048,), _f32),
        pltpu.VMEM((2048,), _f32),
        pltpu.SemaphoreType.DMA,
        pltpu.SemaphoreType.DMA,
        pltpu.SemaphoreType.DMA,
        pltpu.SemaphoreType.DMA,
        pltpu.SemaphoreType.DMA,
        pltpu.SemaphoreType.DMA,
        pltpu.SemaphoreType.DMA,
        pltpu.SemaphoreType.DMA,
        pltpu.SemaphoreType.DMA,
        pltpu.SemaphoreType.DMA,
    ],
)(_agg_body)


def _pool_body(z4, b2d, sums_out, cnts_out, sa0, sa1, sa2, sa3, cacc, ibuf,
               rows, onesb, zbp, zbc):
    c = lax.axis_index("c")
    s = lax.axis_index("s")
    wid = c * NS + s
    saccs = (sa0, sa1, sa2, sa3)
    _fill(onesb, 8, 1.0)
    _fill2d(zbp, 16, 0.0)
    for q in range(4):
        pltpu.sync_copy(zbp, saccs[q].at[pl.ds(s * 16, 16)])

    @pl.when(s == 0)
    def _():
        _fill(zbc, 16, 0.0)
        pltpu.sync_copy(zbc, cacc)

    plsc.subcore_barrier()

    def chunk(k, _):
        cid = wid + 32 * k

        @pl.when(cid < 80)
        def _():
            pltpu.sync_copy(b2d.at[cid], ibuf.at[k])
            for q in range(4):
                pltpu.sync_copy(
                    z4.at[pl.ds(cid * 128, 128), pl.ds(q * 128, 128)], rows)
                pltpu.sync_copy(rows, saccs[q].at[ibuf.at[k]], add=True)
            pltpu.sync_copy(onesb, cacc.at[ibuf.at[k]], add=True)

        return 0

    lax.fori_loop(0, 3, chunk, 0)
    plsc.subcore_barrier()
    for q in range(4):
        pltpu.sync_copy(saccs[q].at[pl.ds(s * 16, 16)],
                        sums_out.at[c, q, pl.ds(s * 16, 16)])

    @pl.when(s == 0)
    def _():
        pltpu.sync_copy(cacc, cnts_out.at[c])


_pool_call = functools.partial(
    pl.kernel,
    out_type=(jax.ShapeDtypeStruct((NC, NBLK, GP, CB), _f32),
              jax.ShapeDtypeStruct((NC, GP), _f32)),
    mesh=_sc_mesh,
    scratch_types=[
        pltpu.VMEM_SHARED((GP, CB), _f32),
        pltpu.VMEM_SHARED((GP, CB), _f32),
        pltpu.VMEM_SHARED((GP, CB), _f32),
        pltpu.VMEM_SHARED((GP, CB), _f32),
        pltpu.VMEM_SHARED((GP,), _f32),
        pltpu.VMEM((3, 128), _i32),
        pltpu.VMEM((128, CB), _f32),
        pltpu.VMEM((128,), _f32),
        pltpu.VMEM((16, CB), _f32),
        pltpu.VMEM((GP,), _f32),
    ],
)(_pool_body)


def _mm_kernel(x_ref, w_ref, g0_ref, g1_ref, o_ref):
    dinv = lax.rsqrt(g0_ref[...] + g1_ref[...] + 1.0)
    h = jnp.dot(x_ref[...], w_ref[...], preferred_element_type=_f32)
    o_ref[...] = h * dinv


def _mm(xp, w, g0, g1):
    din = xp.shape[1]
    return pl.pallas_call(
        _mm_kernel,
        grid=(NP // RB, NBLK),
        in_specs=[
            pl.BlockSpec((RB, din), lambda r, c: (r, 0)),
            pl.BlockSpec((din, CB), lambda r, c: (0, c)),
            pl.BlockSpec((RB, 1), lambda r, c: (r, 0)),
            pl.BlockSpec((RB, 1), lambda r, c: (r, 0)),
        ],
        out_specs=pl.BlockSpec((RB, CB), lambda r, c: (c * (NP // RB) + r, 0)),
        out_shape=jax.ShapeDtypeStruct((NBLK * NP, CB), _f32),
    )(xp, w, g0, g1)


def _comb_kernel(a_ref, h_ref, g0_ref, g1_ref, b_ref, o_ref):
    dinv = lax.rsqrt(g0_ref[...] + g1_ref[...] + 1.0)
    o_ref[...] = jnp.maximum(
        dinv * (a_ref[...] + h_ref[...]) + b_ref[0:1, :], 0.0)


def _comb(agg, hp, g0, g1, b4):
    return pl.pallas_call(
        _comb_kernel,
        grid=(NP // RB, NBLK),
        in_specs=[
            pl.BlockSpec((RB, CB), lambda r, c: (c * (NP // RB) + r, 0)),
            pl.BlockSpec((RB, CB), lambda r, c: (c * (NP // RB) + r, 0)),
            pl.BlockSpec((RB, 1), lambda r, c: (r, 0)),
            pl.BlockSpec((RB, 1), lambda r, c: (r, 0)),
            pl.BlockSpec((8, CB), lambda r, c: (c, 0)),
        ],
        out_specs=pl.BlockSpec((RB, CB), lambda r, c: (r, c)),
        out_shape=jax.ShapeDtypeStruct((NP, H), _f32),
    )(agg, hp, g0, g1, b4)


def _fin_kernel(s0_ref, s1_ref, c0_ref, c1_ref, wc_ref, bc_ref, o_ref):
    cnt = jnp.maximum(c0_ref[...] + c1_ref[...], 1.0)
    pooled = (s0_ref[...] + s1_ref[...]) / cnt
    o_ref[...] = jnp.dot(pooled, wc_ref[...],
                         preferred_element_type=_f32) + bc_ref[...]


def _fin(s0, s1, c0, c1, wc, bc1):
    return pl.pallas_call(
        _fin_kernel,
        out_shape=jax.ShapeDtypeStruct((G, 4), _f32),
    )(s0, s1, c0, c1, wc, bc1)


def kernel(x, edge_index, edge_attr, batch, W1, b1, W2, b2, W3, b3, Wc, bc):
    src = edge_index[0]
    dst = edge_index[1]
    pad = EP - E
    s2 = jnp.concatenate([src, jnp.zeros((pad,), _i32)]).reshape(EC, 128)

    d2 = jnp.concatenate([dst, jnp.full((pad,), NP - 1, _i32)]).reshape(EC, 128)
    wp = jnp.concatenate([edge_attr, jnp.zeros((pad,), _f32)])
    w2 = wp.reshape(EC, 128)
    s2off = (s2[None, :, :]
             + (jnp.arange(NBLK, dtype=_i32) * NP)[:, None, None])
    edata = jnp.concatenate(
        [s2off, jnp.broadcast_to(d2[None], (NBLK, EC, 128))], axis=2)
    w16 = jnp.broadcast_to(wp[:, None], (EP, 16)).reshape(EC, 2048)
    xp = jnp.concatenate([x, jnp.zeros((NP - N, D_IN), _f32)])
    b2d = jnp.concatenate([batch, jnp.full((NP - N,), G, _i32)]).reshape(80, 128)

    deg = _deg_call(d2, w2)
    g0 = deg[0].reshape(NP, 1)
    g1 = deg[1].reshape(NP, 1)

    def _brd(b):
        return jnp.broadcast_to(b.reshape(NBLK, 1, CB),
                                (NBLK, 8, CB)).reshape(NBLK * 8, CB)

    b1r = _brd(b1)
    b2r = _brd(b2)
    b3r = _brd(b3)

    hp1 = _mm(xp, W1, g0, g1)
    ag1 = _agg_call(hp1, edata, w16)
    z2 = _comb(ag1, hp1, g0, g1, b1r)

    hp2 = _mm(z2, W2, g0, g1)
    ag2 = _agg_call(hp2, edata, w16)
    z3 = _comb(ag2, hp2, g0, g1, b2r)

    hp3 = _mm(z3, W3, g0, g1)
    ag3 = _agg_call(hp3, edata, w16)
    z4 = _comb(ag3, hp3, g0, g1, b3r)

    sums, cnts = _pool_call(z4, b2d)
    sums = sums.transpose(0, 2, 1, 3).reshape(NC, GP, H)
    s0 = sums[0, :G]
    s1 = sums[1, :G]
    c0 = cnts[0, :G].reshape(G, 1)
    c1 = cnts[1, :G].reshape(G, 1)
    return _fin(s0, s1, c0, c1, Wc, bc.reshape(1, 4))


# speedup vs baseline: 1.0538x; 1.0431x over previous
import functools

import jax
import jax.numpy as jnp
from jax import lax
from jax.experimental import pallas as pl
from jax.experimental.pallas import tpu as pltpu
from jax.experimental.pallas import tpu_sc as plsc

N = 10000
NP = 10240
E = 160000
EP = 163840
EC = 1280
G = 128
GP = 256
D_IN = 256
H = 512
CB = 128
NBLK = 4
NC = 2
NS = 16
RB = 640

_f32 = jnp.float32
_i32 = jnp.int32

_sc_mesh = plsc.VectorSubcoreMesh(core_axis_name="c", subcore_axis_name="s")


def _fill(ref, n16, value):
    v = jnp.full((16,), value, dtype=ref.dtype)

    def body(i, _):
        ref[pl.ds(i * 16, 16)] = v
        return 0

    lax.fori_loop(0, n16, body, 0)


def _fill2d(ref, rows, value):
    cols = ref.shape[1]
    v = jnp.full((16,), value, dtype=ref.dtype)

    def body(i, _):
        for t in range(cols // 16):
            ref[i, pl.ds(t * 16, 16)] = v
        return 0

    lax.fori_loop(0, rows, body, 0)


def _deg_body(d2, w2, deg_out, dacc, ibuf, wbuf, zb):
    c = lax.axis_index("c")
    s = lax.axis_index("s")
    _fill(zb, 40, 0.0)
    pltpu.sync_copy(zb, dacc.at[pl.ds(s * 640, 640)])
    plsc.subcore_barrier()
    first = c * 640 + s * 40
    pltpu.sync_copy(d2.at[pl.ds(first, 40)], ibuf)
    pltpu.sync_copy(w2.at[pl.ds(first, 40)], wbuf)

    def chunk(j, _):
        pltpu.sync_copy(wbuf.at[j], dacc.at[ibuf.at[j]], add=True)
        return 0

    lax.fori_loop(0, 40, chunk, 0)
    plsc.subcore_barrier()
    pltpu.sync_copy(dacc.at[pl.ds(s * 640, 640)],
                    deg_out.at[c, pl.ds(s * 640, 640)])


_deg_call = functools.partial(
    pl.kernel,
    out_type=jax.ShapeDtypeStruct((NC, NP), _f32),
    mesh=_sc_mesh,
    scratch_types=[
        pltpu.VMEM_SHARED((NP,), _f32),
        pltpu.VMEM((40, 128), _i32),
        pltpu.VMEM((40, 128), _f32),
        pltpu.VMEM((640,), _f32),
    ],
)(_deg_body)


def _agg_body(hp, s2b, d2, w16, agg_out, acc, wbc0, wbc1, rows0, rows1,
              gring, dring, gsem0, gsem1, wsem0, wsem1, ssem0, ssem1,
              isem0, isem1, isem2, isem3):
    c = lax.axis_index("c")
    s = lax.axis_index("s")
    rows = (rows0, rows1)
    wbcs = (wbc0, wbc1)
    gsems = (gsem0, gsem1)
    wsems = (wsem0, wsem1)
    ssems = (ssem0, ssem1)
    isems = (isem0, isem1, isem2, isem3)

    def fire_idx(blk, first, j, slot, isem):
        pltpu.async_copy(s2b.at[blk, first + j], gring.at[slot], isem)
        pltpu.async_copy(d2.at[first + j], dring.at[slot], isem)

    def drain_idx(isem):
        pltpu.make_async_copy(d2.at[0], gring.at[0], isem).wait()
        pltpu.make_async_copy(d2.at[0], dring.at[0], isem).wait()

    def fire_rows(first, j, slot, p):
        pltpu.async_copy(hp.at[gring.at[slot]], rows[p], gsems[p])
        pltpu.async_copy(w16.at[first + j], wbcs[p], wsems[p])

    def drain_rows(p):
        pltpu.make_async_copy(hp.at[pl.ds(0, 128)], rows[p],
                              gsems[p]).wait()
        pltpu.make_async_copy(w16.at[0], wbcs[p], wsems[p]).wait()

    def drain_scat(p):
        pltpu.make_async_copy(hp.at[pl.ds(0, 128)], rows[p],
                              ssems[p]).wait()

    def scale(rr, wbc, lo, hi):
        def body(i, _):
            for u in range(2):
                e = 2 * i + u
                wv = wbc[pl.ds(e * 16, 16)]
                for t in range(CB // 16):
                    rr[e, pl.ds(t * 16, 16)] = rr[e, pl.ds(t * 16, 16)] * wv
            return 0

        lax.fori_loop(lo // 2, hi // 2, body, 0)

    for blk_i in range(2):
        blk = c * 2 + blk_i
        base = blk * NP
        _fill2d(rows0, 128, 0.0)
        for k in range(5):
            pltpu.sync_copy(rows0, acc.at[pl.ds(s * 640 + k * 128, 128)])
        plsc.subcore_barrier()

        for half in range(2):
            first = s * 80 + half * 40
            fire_idx(blk, first, 0, 0, isem0)
            fire_idx(blk, first, 1, 1, isem1)
            drain_idx(isem0)
            fire_rows(first, 0, 0, 0)

            def quad(k, _):
                for t in range(4):
                    j = 4 * k + t
                    p = t & 1

                    drain_rows(p)
                    scale(rows[p], wbcs[p], 0, 64)
                    if t == 0:
                        @pl.when(k > 0)
                        def _():
                            drain_scat(1 - p)
                    else:
                        drain_scat(1 - p)
                    if t == 3:
                        @pl.when(k < 9)
                        def _():
                            drain_idx(isems[(t + 1) & 3])
                            fire_rows(first, j + 1, (t + 1) & 3, 1 - p)
                            fire_idx(blk, first, j + 2, (t + 2) & 3,
                                     isems[(t + 2) & 3])
                    else:
                        drain_idx(isems[(t + 1) & 3])
                        fire_rows(first, j + 1, (t + 1) & 3, 1 - p)

                        if t == 2:
                            @pl.when(k < 9)
                            def _():
                                fire_idx(blk, first, j + 2, (t + 2) & 3,
                                         isems[(t + 2) & 3])
                        else:
                            fire_idx(blk, first, j + 2, (t + 2) & 3,
                                     isems[(t + 2) & 3])
                    scale(rows[p], wbcs[p], 64, 128)
                    pltpu.async_copy(rows[p], acc.at[dring.at[lax.rem(j, 4)]],
                                     ssems[p], add=True)
                return 0

            lax.fori_loop(0, 10, quad, 0)
            drain_scat(1)
        plsc.subcore_barrier()
        pltpu.sync_copy(acc.at[pl.ds(s * 640, 640)],
                        agg_out.at[pl.ds(base + s * 640, 640)])
        plsc.subcore_barrier()


_agg_call = functools.partial(
    pl.kernel,
    out_type=jax.ShapeDtypeStruct((NBLK * NP, CB), _f32),
    mesh=_sc_mesh,
    scratch_types=[
        pltpu.VMEM_SHARED((NP, CB), _f32),
        pltpu.VMEM((2048,), _f32),
        pltpu.VMEM((2048,), _f32),
        pltpu.VMEM((128, CB), _f32),
        pltpu.VMEM((128, CB), _f32),
        pltpu.VMEM((4, 128), _i32),
        pltpu.VMEM((4, 128), _i32),
        pltpu.SemaphoreType.DMA,
        pltpu.SemaphoreType.DMA,
        pltpu.SemaphoreType.DMA,
        pltpu.SemaphoreType.DMA,
        pltpu.SemaphoreType.DMA,
        pltpu.SemaphoreType.DMA,
        pltpu.SemaphoreType.DMA,
        pltpu.SemaphoreType.DMA,
        pltpu.SemaphoreType.DMA,
        pltpu.SemaphoreType.DMA,
    ],
)(_agg_body)


def _pool_body(z4, b2d, sums_out, cnts_out, sa0, sa1, sa2, sa3, cacc, ibuf,
               rows, onesb, zbp, zbc):
    c = lax.axis_index("c")
    s = lax.axis_index("s")
    wid = c * NS + s
    saccs = (sa0, sa1, sa2, sa3)
    _fill(onesb, 8, 1.0)
    _fill2d(zbp, 16, 0.0)
    for q in range(4):
        pltpu.sync_copy(zbp, saccs[q].at[pl.ds(s * 16, 16)])

    @pl.when(s == 0)
    def _():
        _fill(zbc, 16, 0.0)
        pltpu.sync_copy(zbc, cacc)

    plsc.subcore_barrier()

    def chunk(k, _):
        cid = wid + 32 * k

        @pl.when(cid < 80)
        def _():
            pltpu.sync_copy(b2d.at[cid], ibuf.at[k])
            for q in range(4):
                pltpu.sync_copy(
                    z4.at[pl.ds(cid * 128, 128), pl.ds(q * 128, 128)], rows)
                pltpu.sync_copy(rows, saccs[q].at[ibuf.at[k]], add=True)
            pltpu.sync_copy(onesb, cacc.at[ibuf.at[k]], add=True)

        return 0

    lax.fori_loop(0, 3, chunk, 0)
    plsc.subcore_barrier()
    for q in range(4):
        pltpu.sync_copy(saccs[q].at[pl.ds(s * 16, 16)],
                        sums_out.at[c, q, pl.ds(s * 16, 16)])

    @pl.when(s == 0)
    def _():
        pltpu.sync_copy(cacc, cnts_out.at[c])


_pool_call = functools.partial(
    pl.kernel,
    out_type=(jax.ShapeDtypeStruct((NC, NBLK, GP, CB), _f32),
              jax.ShapeDtypeStruct((NC, GP), _f32)),
    mesh=_sc_mesh,
    scratch_types=[
        pltpu.VMEM_SHARED((GP, CB), _f32),
        pltpu.VMEM_SHARED((GP, CB), _f32),
        pltpu.VMEM_SHARED((GP, CB), _f32),
        pltpu.VMEM_SHARED((GP, CB), _f32),
        pltpu.VMEM_SHARED((GP,), _f32),
        pltpu.VMEM((3, 128), _i32),
        pltpu.VMEM((128, CB), _f32),
        pltpu.VMEM((128,), _f32),
        pltpu.VMEM((16, CB), _f32),
        pltpu.VMEM((GP,), _f32),
    ],
)(_pool_body)


def _mm_kernel(x_ref, w_ref, g0_ref, g1_ref, o_ref):
    dinv = lax.rsqrt(g0_ref[...] + g1_ref[...] + 1.0)
    h = jnp.dot(x_ref[...], w_ref[...], preferred_element_type=_f32)
    o_ref[...] = h * dinv


def _mm(xp, w, g0, g1):
    din = xp.shape[1]
    return pl.pallas_call(
        _mm_kernel,
        grid=(NP // RB, NBLK),
        in_specs=[
            pl.BlockSpec((RB, din), lambda r, c: (r, 0)),
            pl.BlockSpec((din, CB), lambda r, c: (0, c)),
            pl.BlockSpec((RB, 1), lambda r, c: (r, 0)),
            pl.BlockSpec((RB, 1), lambda r, c: (r, 0)),
        ],
        out_specs=pl.BlockSpec((RB, CB), lambda r, c: (c * (NP // RB) + r, 0)),
        out_shape=jax.ShapeDtypeStruct((NBLK * NP, CB), _f32),
    )(xp, w, g0, g1)


def _comb_kernel(a_ref, h_ref, g0_ref, g1_ref, b_ref, o_ref):
    dinv = lax.rsqrt(g0_ref[...] + g1_ref[...] + 1.0)
    o_ref[...] = jnp.maximum(
        dinv * (a_ref[...] + h_ref[...]) + b_ref[0:1, :], 0.0)


def _comb(agg, hp, g0, g1, b4):
    return pl.pallas_call(
        _comb_kernel,
        grid=(NP // RB, NBLK),
        in_specs=[
            pl.BlockSpec((RB, CB), lambda r, c: (c * (NP // RB) + r, 0)),
            pl.BlockSpec((RB, CB), lambda r, c: (c * (NP // RB) + r, 0)),
            pl.BlockSpec((RB, 1), lambda r, c: (r, 0)),
            pl.BlockSpec((RB, 1), lambda r, c: (r, 0)),
            pl.BlockSpec((8, CB), lambda r, c: (c, 0)),
        ],
        out_specs=pl.BlockSpec((RB, CB), lambda r, c: (r, c)),
        out_shape=jax.ShapeDtypeStruct((NP, H), _f32),
    )(agg, hp, g0, g1, b4)


def _fin_kernel(s0_ref, s1_ref, c0_ref, c1_ref, wc_ref, bc_ref, o_ref):
    cnt = jnp.maximum(c0_ref[...] + c1_ref[...], 1.0)
    pooled = (s0_ref[...] + s1_ref[...]) / cnt
    o_ref[...] = jnp.dot(pooled, wc_ref[...],
                         preferred_element_type=_f32) + bc_ref[...]


def _fin(s0, s1, c0, c1, wc, bc1):
    return pl.pallas_call(
        _fin_kernel,
        out_shape=jax.ShapeDtypeStruct((G, 4), _f32),
    )(s0, s1, c0, c1, wc, bc1)


def kernel(x, edge_index, edge_attr, batch, W1, b1, W2, b2, W3, b3, Wc, bc):
    src = edge_index[0]
    dst = edge_index[1]
    pad = EP - E
    s2 = jnp.concatenate([src, jnp.zeros((pad,), _i32)]).reshape(EC, 128)
    s2b = (s2[None, :, :]
           + (jnp.arange(NBLK, dtype=_i32) * NP)[:, None, None])
    d2 = jnp.concatenate([dst, jnp.full((pad,), NP - 1, _i32)]).reshape(EC, 128)
    wp = jnp.concatenate([edge_attr, jnp.zeros((pad,), _f32)])
    w2 = wp.reshape(EC, 128)
    w16 = jnp.broadcast_to(wp[:, None], (EP, 16)).reshape(EC, 2048)
    xp = jnp.concatenate([x, jnp.zeros((NP - N, D_IN), _f32)])
    b2d = jnp.concatenate([batch, jnp.full((NP - N,), G, _i32)]).reshape(80, 128)

    deg = _deg_call(d2, w2)
    g0 = deg[0].reshape(NP, 1)
    g1 = deg[1].reshape(NP, 1)

    def _brd(b):
        return jnp.broadcast_to(b.reshape(NBLK, 1, CB),
                                (NBLK, 8, CB)).reshape(NBLK * 8, CB)

    b1r = _brd(b1)
    b2r = _brd(b2)
    b3r = _brd(b3)

    hp1 = _mm(xp, W1, g0, g1)
    ag1 = _agg_call(hp1, s2b, d2, w16)
    z2 = _comb(ag1, hp1, g0, g1, b1r)

    hp2 = _mm(z2, W2, g0, g1)
    ag2 = _agg_call(hp2, s2b, d2, w16)
    z3 = _comb(ag2, hp2, g0, g1, b2r)

    hp3 = _mm(z3, W3, g0, g1)
    ag3 = _agg_call(hp3, s2b, d2, w16)
    z4 = _comb(ag3, hp3, g0, g1, b3r)

    sums, cnts = _pool_call(z4, b2d)
    sums = sums.transpose(0, 2, 1, 3).reshape(NC, GP, H)
    s0 = sums[0, :G]
    s1 = sums[1, :G]
    c0 = cnts[0, :G].reshape(G, 1)
    c1 = cnts[1, :G].reshape(G, 1)
    return _fin(s0, s1, c0, c1, Wc, bc.reshape(1, 4))


# speedup vs baseline: 1.1172x; 1.0602x over previous
import functools

import jax
import jax.numpy as jnp
from jax import lax
from jax.experimental import pallas as pl
from jax.experimental.pallas import tpu as pltpu
from jax.experimental.pallas import tpu_sc as plsc

N = 10000
NP = 10240
E = 160000
EP = 163840
EC = 1280
G = 128
GP = 256
D_IN = 256
H = 512
CB = 128
NBLK = 4
NC = 2
NS = 16
RB = 640

_f32 = jnp.float32
_i32 = jnp.int32

_sc_mesh = plsc.VectorSubcoreMesh(core_axis_name="c", subcore_axis_name="s")


def _fill(ref, n16, value):
    v = jnp.full((16,), value, dtype=ref.dtype)

    def body(i, _):
        ref[pl.ds(i * 16, 16)] = v
        return 0

    lax.fori_loop(0, n16, body, 0)


def _fill2d(ref, rows, value):
    cols = ref.shape[1]
    v = jnp.full((16,), value, dtype=ref.dtype)

    def body(i, _):
        for t in range(cols // 16):
            ref[i, pl.ds(t * 16, 16)] = v
        return 0

    lax.fori_loop(0, rows, body, 0)


def _deg_body(d2, w2, deg_out, dacc, ibuf, wbuf, zb):
    c = lax.axis_index("c")
    s = lax.axis_index("s")
    _fill(zb, 40, 0.0)
    pltpu.sync_copy(zb, dacc.at[pl.ds(s * 640, 640)])
    plsc.subcore_barrier()
    first = c * 640 + s * 40
    pltpu.sync_copy(d2.at[pl.ds(first, 40)], ibuf)
    pltpu.sync_copy(w2.at[pl.ds(first, 40)], wbuf)

    def chunk(j, _):
        pltpu.sync_copy(wbuf.at[j], dacc.at[ibuf.at[j]], add=True)
        return 0

    lax.fori_loop(0, 40, chunk, 0)
    plsc.subcore_barrier()
    pltpu.sync_copy(dacc.at[pl.ds(s * 640, 640)],
                    deg_out.at[c, pl.ds(s * 640, 640)])


_deg_call = functools.partial(
    pl.kernel,
    out_type=jax.ShapeDtypeStruct((NC, NP), _f32),
    mesh=_sc_mesh,
    scratch_types=[
        pltpu.VMEM_SHARED((NP,), _f32),
        pltpu.VMEM((40, 128), _i32),
        pltpu.VMEM((40, 128), _f32),
        pltpu.VMEM((640,), _f32),
    ],
)(_deg_body)


def _agg_body(hp, s2b, d2, w16, agg_out, acc, wbc0, wbc1, rows0, rows1,
              gring, dring, gsem0, gsem1, wsem0, wsem1, ssem0, ssem1,
              isem0, isem1, isem2, isem3):
    c = lax.axis_index("c")
    s = lax.axis_index("s")
    rows = (rows0, rows1)
    wbcs = (wbc0, wbc1)
    gsems = (gsem0, gsem1)
    wsems = (wsem0, wsem1)
    ssems = (ssem0, ssem1)
    isems = (isem0, isem1, isem2, isem3)

    def fire_idx(blk, first, j, slot, isem):
        pltpu.async_copy(s2b.at[blk, first + j], gring.at[slot], isem)
        pltpu.async_copy(d2.at[first + j], dring.at[slot], isem)

    def drain_idx(isem):
        pltpu.make_async_copy(d2.at[0], gring.at[0], isem).wait()
        pltpu.make_async_copy(d2.at[0], dring.at[0], isem).wait()

    def fire_rows(first, j, slot, p):
        pltpu.async_copy(hp.at[gring.at[slot]], rows[p], gsems[p])
        pltpu.async_copy(w16.at[first + j], wbcs[p], wsems[p])

    def drain_rows(p):
        pltpu.make_async_copy(hp.at[pl.ds(0, 128)], rows[p],
                              gsems[p]).wait()
        pltpu.make_async_copy(w16.at[0], wbcs[p], wsems[p]).wait()

    def drain_scat(p):
        pltpu.make_async_copy(hp.at[pl.ds(0, 128)], rows[p],
                              ssems[p]).wait()

    def scale(rr, wbc, lo, hi):
        def body(i, _):
            for u in range(2):
                e = 2 * i + u
                wv = wbc[pl.ds(e * 16, 16)]
                for t in range(CB // 16):
                    rr[e, pl.ds(t * 16, 16)] = rr[e, pl.ds(t * 16, 16)] * wv
            return 0

        lax.fori_loop(lo // 2, hi // 2, body, 0)

    for blk_i in range(2):
        blk = c * 2 + blk_i
        base = blk * NP
        _fill2d(rows0, 128, 0.0)
        for k in range(5):
            pltpu.sync_copy(rows0, acc.at[pl.ds(s * 640 + k * 128, 128)])
        plsc.subcore_barrier()

        for half in range(2):
            first = s * 80 + half * 40
            fire_idx(blk, first, 0, 0, isem0)
            fire_idx(blk, first, 1, 1, isem1)
            drain_idx(isem0)
            fire_rows(first, 0, 0, 0)

            def quad(k, _):
                for t in range(4):
                    j = 4 * k + t
                    p = t & 1

                    drain_rows(p)
                    scale(rows[p], wbcs[p], 0, 64)
                    if t == 0:
                        @pl.when(k > 0)
                        def _():
                            drain_scat(1 - p)
                    else:
                        drain_scat(1 - p)
                    if t == 3:
                        @pl.when(k < 9)
                        def _():
                            drain_idx(isems[(t + 1) & 3])
                            fire_rows(first, j + 1, (t + 1) & 3, 1 - p)
                            fire_idx(blk, first, j + 2, (t + 2) & 3,
                                     isems[(t + 2) & 3])
                    else:
                        drain_idx(isems[(t + 1) & 3])
                        fire_rows(first, j + 1, (t + 1) & 3, 1 - p)

                        if t == 2:
                            @pl.when(k < 9)
                            def _():
                                fire_idx(blk, first, j + 2, (t + 2) & 3,
                                         isems[(t + 2) & 3])
                        else:
                            fire_idx(blk, first, j + 2, (t + 2) & 3,
                                     isems[(t + 2) & 3])
                    scale(rows[p], wbcs[p], 64, 128)
                    pltpu.async_copy(rows[p], acc.at[dring.at[lax.rem(j, 4)]],
                                     ssems[p], add=True)
                return 0

            lax.fori_loop(0, 10, quad, 0)
            drain_scat(1)
        plsc.subcore_barrier()
        pltpu.sync_copy(acc.at[pl.ds(s * 640, 640)],
                        agg_out.at[pl.ds(base + s * 640, 640)])
        plsc.subcore_barrier()


_agg_call = functools.partial(
    pl.kernel,
    out_type=jax.ShapeDtypeStruct((NBLK * NP, CB), _f32),
    mesh=_sc_mesh,
    scratch_types=[
        pltpu.VMEM_SHARED((NP, CB), _f32),
        pltpu.VMEM((2048,), _f32),
        pltpu.VMEM((2048,), _f32),
        pltpu.VMEM((128, CB), _f32),
        pltpu.VMEM((128, CB), _f32),
        pltpu.VMEM((4, 128), _i32),
        pltpu.VMEM((4, 128), _i32),
        pltpu.SemaphoreType.DMA,
        pltpu.SemaphoreType.DMA,
        pltpu.SemaphoreType.DMA,
        pltpu.SemaphoreType.DMA,
        pltpu.SemaphoreType.DMA,
        pltpu.SemaphoreType.DMA,
        pltpu.SemaphoreType.DMA,
        pltpu.SemaphoreType.DMA,
        pltpu.SemaphoreType.DMA,
        pltpu.SemaphoreType.DMA,
    ],
)(_agg_body)


def _pool_body(z4, b2d, sums_out, cnts_out, sa0, sa1, sa2, sa3, cacc, ibuf,
               rows, onesb, zbp, zbc):
    c = lax.axis_index("c")
    s = lax.axis_index("s")
    wid = c * NS + s
    saccs = (sa0, sa1, sa2, sa3)
    _fill(onesb, 8, 1.0)
    _fill2d(zbp, 16, 0.0)
    for q in range(4):
        pltpu.sync_copy(zbp, saccs[q].at[pl.ds(s * 16, 16)])

    @pl.when(s == 0)
    def _():
        _fill(zbc, 16, 0.0)
        pltpu.sync_copy(zbc, cacc)

    plsc.subcore_barrier()

    def chunk(k, _):
        cid = wid + 32 * k

        @pl.when(cid < 80)
        def _():
            pltpu.sync_copy(b2d.at[cid], ibuf.at[k])
            for q in range(4):
                pltpu.sync_copy(
                    z4.at[pl.ds(cid * 128, 128), pl.ds(q * 128, 128)], rows)
                pltpu.sync_copy(rows, saccs[q].at[ibuf.at[k]], add=True)
            pltpu.sync_copy(onesb, cacc.at[ibuf.at[k]], add=True)

        return 0

    lax.fori_loop(0, 3, chunk, 0)
    plsc.subcore_barrier()
    for q in range(4):
        pltpu.sync_copy(saccs[q].at[pl.ds(s * 16, 16)],
                        sums_out.at[c, q, pl.ds(s * 16, 16)])

    @pl.when(s == 0)
    def _():
        pltpu.sync_copy(cacc, cnts_out.at[c])


_pool_call = functools.partial(
    pl.kernel,
    out_type=(jax.ShapeDtypeStruct((NC, NBLK, GP, CB), _f32),
              jax.ShapeDtypeStruct((NC, GP), _f32)),
    mesh=_sc_mesh,
    scratch_types=[
        pltpu.VMEM_SHARED((GP, CB), _f32),
        pltpu.VMEM_SHARED((GP, CB), _f32),
        pltpu.VMEM_SHARED((GP, CB), _f32),
        pltpu.VMEM_SHARED((GP, CB), _f32),
        pltpu.VMEM_SHARED((GP,), _f32),
        pltpu.VMEM((3, 128), _i32),
        pltpu.VMEM((128, CB), _f32),
        pltpu.VMEM((128,), _f32),
        pltpu.VMEM((16, CB), _f32),
        pltpu.VMEM((GP,), _f32),
    ],
)(_pool_body)


def _mm_kernel(x_ref, w_ref, g0_ref, g1_ref, o_ref):
    dinv = lax.rsqrt(g0_ref[...] + g1_ref[...] + 1.0)
    h = jnp.dot(x_ref[...], w_ref[...], preferred_element_type=_f32)
    o_ref[...] = h * dinv


def _mm(xp, w, g0, g1):
    din = xp.shape[1]
    return pl.pallas_call(
        _mm_kernel,
        grid=(NP // RB, NBLK),
        in_specs=[
            pl.BlockSpec((RB, din), lambda r, c: (r, 0)),
            pl.BlockSpec((din, CB), lambda r, c: (0, c)),
            pl.BlockSpec((RB, 1), lambda r, c: (r, 0)),
            pl.BlockSpec((RB, 1), lambda r, c: (r, 0)),
        ],
        out_specs=pl.BlockSpec((RB, CB), lambda r, c: (c * (NP // RB) + r, 0)),
        out_shape=jax.ShapeDtypeStruct((NBLK * NP, CB), _f32),
    )(xp, w, g0, g1)


def _mmf_kernel(a_ref, h_ref, g0_ref, g1_ref, b_ref, w_ref, o_ref):
    dinv = lax.rsqrt(g0_ref[...] + g1_ref[...] + 1.0)
    a = jnp.concatenate([a_ref[q] for q in range(NBLK)], axis=1)
    h = jnp.concatenate([h_ref[q] for q in range(NBLK)], axis=1)
    z = jnp.maximum(dinv * (a + h) + b_ref[0:1, :], 0.0)
    h2 = jnp.dot(z, w_ref[...], preferred_element_type=_f32) * dinv
    for q in range(NBLK):
        o_ref[q] = h2[:, q * CB:(q + 1) * CB]


def _mmf(agg, hp, g0, g1, b8, w):
    out = pl.pallas_call(
        _mmf_kernel,
        grid=(NP // RB,),
        in_specs=[
            pl.BlockSpec((NBLK, RB, CB), lambda r: (0, r, 0)),
            pl.BlockSpec((NBLK, RB, CB), lambda r: (0, r, 0)),
            pl.BlockSpec((RB, 1), lambda r: (r, 0)),
            pl.BlockSpec((RB, 1), lambda r: (r, 0)),
            pl.BlockSpec((8, H), lambda r: (0, 0)),
            pl.BlockSpec((H, H), lambda r: (0, 0)),
        ],
        out_specs=pl.BlockSpec((NBLK, RB, CB), lambda r: (0, r, 0)),
        out_shape=jax.ShapeDtypeStruct((NBLK, NP, CB), _f32),
    )(agg.reshape(NBLK, NP, CB), hp.reshape(NBLK, NP, CB), g0, g1, b8, w)
    return out.reshape(NBLK * NP, CB)


def _comb_kernel(a_ref, h_ref, g0_ref, g1_ref, b_ref, o_ref):
    dinv = lax.rsqrt(g0_ref[...] + g1_ref[...] + 1.0)
    o_ref[...] = jnp.maximum(
        dinv * (a_ref[...] + h_ref[...]) + b_ref[0:1, :], 0.0)


def _comb(agg, hp, g0, g1, b4):
    return pl.pallas_call(
        _comb_kernel,
        grid=(NP // RB, NBLK),
        in_specs=[
            pl.BlockSpec((RB, CB), lambda r, c: (c * (NP // RB) + r, 0)),
            pl.BlockSpec((RB, CB), lambda r, c: (c * (NP // RB) + r, 0)),
            pl.BlockSpec((RB, 1), lambda r, c: (r, 0)),
            pl.BlockSpec((RB, 1), lambda r, c: (r, 0)),
            pl.BlockSpec((8, CB), lambda r, c: (c, 0)),
        ],
        out_specs=pl.BlockSpec((RB, CB), lambda r, c: (r, c)),
        out_shape=jax.ShapeDtypeStruct((NP, H), _f32),
    )(agg, hp, g0, g1, b4)


def _fin_kernel(s0_ref, s1_ref, c0_ref, c1_ref, wc_ref, bc_ref, o_ref):
    cnt = jnp.maximum(c0_ref[...] + c1_ref[...], 1.0)
    pooled = (s0_ref[...] + s1_ref[...]) / cnt
    o_ref[...] = jnp.dot(pooled, wc_ref[...],
                         preferred_element_type=_f32) + bc_ref[...]


def _fin(s0, s1, c0, c1, wc, bc1):
    return pl.pallas_call(
        _fin_kernel,
        out_shape=jax.ShapeDtypeStruct((G, 4), _f32),
    )(s0, s1, c0, c1, wc, bc1)


def kernel(x, edge_index, edge_attr, batch, W1, b1, W2, b2, W3, b3, Wc, bc):
    src = edge_index[0]
    dst = edge_index[1]
    pad = EP - E
    s2 = jnp.concatenate([src, jnp.zeros((pad,), _i32)]).reshape(EC, 128)
    s2b = (s2[None, :, :]
           + (jnp.arange(NBLK, dtype=_i32) * NP)[:, None, None])
    d2 = jnp.concatenate([dst, jnp.full((pad,), NP - 1, _i32)]).reshape(EC, 128)
    wp = jnp.concatenate([edge_attr, jnp.zeros((pad,), _f32)])
    w2 = wp.reshape(EC, 128)
    w16 = jnp.broadcast_to(wp[:, None], (EP, 16)).reshape(EC, 2048)
    xp = jnp.concatenate([x, jnp.zeros((NP - N, D_IN), _f32)])
    b2d = jnp.concatenate([batch, jnp.full((NP - N,), G, _i32)]).reshape(80, 128)

    deg = _deg_call(d2, w2)
    g0 = deg[0].reshape(NP, 1)
    g1 = deg[1].reshape(NP, 1)

    def _brd(b):
        return jnp.broadcast_to(b.reshape(NBLK, 1, CB),
                                (NBLK, 8, CB)).reshape(NBLK * 8, CB)

    b1r = _brd(b1)
    b2r = _brd(b2)
    b3r = _brd(b3)

    def _brd8(b):
        return jnp.broadcast_to(b.reshape(1, H), (8, H))

    hp1 = _mm(xp, W1, g0, g1)
    ag1 = _agg_call(hp1, s2b, d2, w16)
    hp2 = _mmf(ag1, hp1, g0, g1, _brd8(b1), W2)
    ag2 = _agg_call(hp2, s2b, d2, w16)
    hp3 = _mmf(ag2, hp2, g0, g1, _brd8(b2), W3)
    ag3 = _agg_call(hp3, s2b, d2, w16)
    z4 = _comb(ag3, hp3, g0, g1, b3r)

    sums, cnts = _pool_call(z4, b2d)
    sums = sums.transpose(0, 2, 1, 3).reshape(NC, GP, H)
    s0 = sums[0, :G]
    s1 = sums[1, :G]
    c0 = cnts[0, :G].reshape(G, 1)
    c1 = cnts[1, :G].reshape(G, 1)
    return _fin(s0, s1, c0, c1, Wc, bc.reshape(1, 4))
